# initial kernel scaffold (unmeasured)
import jax
import jax.numpy as jnp
from jax import lax
from jax.experimental import pallas as pl
from jax.experimental.pallas import tpu as pltpu

B, H, D, BS = 8, 8, 64, 16
SLOTS = 64
PAGES_LOCAL = 64
T = PAGES_LOCAL * BS
NEG = -1e30


def kernel(Q, K, V, bt, lens):
    def body(q_ref, k_ref, v_ref, bt_ref, lens_ref, out_ref,
             o_buf, m_buf, l_buf, send_sems, recv_sems):
        my_x = lax.axis_index("x")
        my_y = lax.axis_index("y")

        barrier = pltpu.get_barrier_semaphore()
        pl.semaphore_signal(
            barrier, inc=1,
            device_id=(1 - my_x, my_y), device_id_type=pl.DeviceIdType.MESH,
        )
        pl.semaphore_wait(barrier, 1)

        bt_v = bt_ref[:]
        lens_v = lens_ref[:].reshape(B, 1)
        slot_i = lax.broadcasted_iota(jnp.int32, (B, SLOTS), 1)
        valid = slot_i < lens_v
        pg_tok = my_x * PAGES_LOCAL + (
            lax.broadcasted_iota(jnp.int32, (1, 1, T), 2) // BS
        )
        eq = (bt_v[:, :, None] == pg_tok) & valid[:, :, None]
        w_tok = jnp.sum(eq.astype(jnp.float32), axis=1)
        bias = jnp.where(w_tok > 0.0, 0.0, NEG)

        kf = k_ref[:].reshape(T, H, D).astype(jnp.bfloat16)
        vf = v_ref[:].reshape(T, H, D).astype(jnp.bfloat16)
        scale = D ** -0.5

        for h in range(H):
            q_h = q_ref[:, 0, h, :].astype(jnp.bfloat16)
            s = lax.dot_general(
                q_h, kf[:, h, :], (((1,), (1,)), ((), ())),
                preferred_element_type=jnp.float32,
            ) * scale + bias
            m_h = jnp.max(s, axis=1, keepdims=True)
            p = jnp.exp(s - m_h) * w_tok
            l_h = jnp.sum(p, axis=1, keepdims=True)
            o_h = lax.dot_general(
                p.astype(jnp.bfloat16), vf[:, h, :],
                (((1,), (0,)), ((), ())),
                preferred_element_type=jnp.float32,
            )
            o_buf[0, h] = o_h
            m_buf[0, h] = m_h
            l_buf[0, h] = l_h

        copies = []
        for i, buf in enumerate((o_buf, m_buf, l_buf)):
            c = pltpu.make_async_remote_copy(
                src_ref=buf.at[0],
                dst_ref=buf.at[1],
                send_sem=send_sems.at[i],
                recv_sem=recv_sems.at[i],
                device_id=(1 - my_x, my_y),
                device_id_type=pl.DeviceIdType.MESH,
            )
            c.start()
            copies.append(c)
        for c in copies:
            c.wait()

        for h in range(H):
            m0, m1 = m_buf[0, h], m_buf[1, h]
            l0, l1 = l_buf[0, h], l_buf[1, h]
            o0, o1 = o_buf[0, h], o_buf[1, h]
            mg = jnp.maximum(m0, m1)
            a0 = jnp.exp(m0 - mg)
            a1 = jnp.exp(m1 - mg)
            lg = a0 * l0 + a1 * l1
            out_ref[:, 0, h, :] = (a0 * o0 + a1 * o1) / lg

    return pl.pallas_call(
        body,
        out_shape=jax.ShapeDtypeStruct((B, 1, H, D), jnp.float32),
        in_specs=[pl.BlockSpec(memory_space=pltpu.VMEM)] * 5,
        out_specs=pl.BlockSpec(memory_space=pltpu.VMEM),
        scratch_shapes=[
            pltpu.VMEM((2, H, B, D), jnp.float32),
            pltpu.VMEM((2, H, B, 1), jnp.float32),
            pltpu.VMEM((2, H, B, 1), jnp.float32),
            pltpu.SemaphoreType.DMA((3,)),
            pltpu.SemaphoreType.DMA((3,)),
        ],
        compiler_params=pltpu.CompilerParams(collective_id=0),
    )(Q, K, V, bt, lens)


# baseline (device time: 17631 ns/iter reference)
import jax
import jax.numpy as jnp
from jax import lax
from jax.experimental import pallas as pl
from jax.experimental.pallas import tpu as pltpu

B, H, D, BS = 8, 8, 64, 16
SLOTS = 64
PAGES_LOCAL = 64
T = PAGES_LOCAL * BS
NEG = -1e30


def kernel(Q, K, V, bt, lens):
    def body(q_ref, k_ref, v_ref, bt_ref, lens_ref, out_ref,
             o_buf, m_buf, l_buf, w_pgT, send_sems, recv_sems):
        my_x = lax.axis_index("x")
        my_y = lax.axis_index("y")

        barrier = pltpu.get_barrier_semaphore()
        pl.semaphore_signal(
            barrier, inc=1,
            device_id=(1 - my_x, my_y), device_id_type=pl.DeviceIdType.MESH,
        )
        pl.semaphore_wait(barrier, 1)

        p_col = my_x * PAGES_LOCAL + lax.broadcasted_iota(
            jnp.int32, (PAGES_LOCAL, 1), 0
        )
        slot_row = lax.broadcasted_iota(jnp.int32, (1, SLOTS), 1)
        for b in range(B):
            bt_row = bt_ref[b : b + 1, :]
            valid_row = slot_row < lens_ref[b]
            cmp = (bt_row == p_col) & valid_row
            w_pgT[:, b : b + 1] = jnp.sum(
                cmp.astype(jnp.float32), axis=1, keepdims=True
            )
        exp_mat = (
            lax.broadcasted_iota(jnp.int32, (PAGES_LOCAL, T), 0)
            == lax.broadcasted_iota(jnp.int32, (PAGES_LOCAL, T), 1) // BS
        ).astype(jnp.float32)
        w_tok = lax.dot_general(
            w_pgT[:], exp_mat, (((0,), (0,)), ((), ())),
            preferred_element_type=jnp.float32,
        )
        bias = jnp.where(w_tok > 0.0, 0.0, NEG)

        kf = k_ref[:].reshape(T, H, D).astype(jnp.bfloat16)
        vf = v_ref[:].reshape(T, H, D).astype(jnp.bfloat16)
        scale = D ** -0.5

        for h in range(H):
            q_h = q_ref[:, 0, h, :].astype(jnp.bfloat16)
            s = lax.dot_general(
                q_h, kf[:, h, :], (((1,), (1,)), ((), ())),
                preferred_element_type=jnp.float32,
            ) * scale + bias
            m_h = jnp.max(s, axis=1, keepdims=True)
            p = jnp.exp(s - m_h) * w_tok
            l_h = jnp.sum(p, axis=1, keepdims=True)
            o_h = lax.dot_general(
                p.astype(jnp.bfloat16), vf[:, h, :],
                (((1,), (0,)), ((), ())),
                preferred_element_type=jnp.float32,
            )
            o_buf[0, h] = o_h
            m_buf[0, h] = m_h
            l_buf[0, h] = l_h

        copies = []
        for i, buf in enumerate((o_buf, m_buf, l_buf)):
            c = pltpu.make_async_remote_copy(
                src_ref=buf.at[0],
                dst_ref=buf.at[1],
                send_sem=send_sems.at[i],
                recv_sem=recv_sems.at[i],
                device_id=(1 - my_x, my_y),
                device_id_type=pl.DeviceIdType.MESH,
            )
            c.start()
            copies.append(c)
        for c in copies:
            c.wait()

        for h in range(H):
            m0, m1 = m_buf[0, h], m_buf[1, h]
            l0, l1 = l_buf[0, h], l_buf[1, h]
            o0, o1 = o_buf[0, h], o_buf[1, h]
            mg = jnp.maximum(m0, m1)
            a0 = jnp.exp(m0 - mg)
            a1 = jnp.exp(m1 - mg)
            lg = a0 * l0 + a1 * l1
            out_ref[:, 0, h, :] = (a0 * o0 + a1 * o1) / lg

    return pl.pallas_call(
        body,
        out_shape=jax.ShapeDtypeStruct((B, 1, H, D), jnp.float32),
        in_specs=[pl.BlockSpec(memory_space=pltpu.VMEM)] * 4
        + [pl.BlockSpec(memory_space=pltpu.SMEM)],
        out_specs=pl.BlockSpec(memory_space=pltpu.VMEM),
        scratch_shapes=[
            pltpu.VMEM((2, H, B, D), jnp.float32),
            pltpu.VMEM((2, H, B, 1), jnp.float32),
            pltpu.VMEM((2, H, B, 1), jnp.float32),
            pltpu.VMEM((PAGES_LOCAL, B), jnp.float32),
            pltpu.SemaphoreType.DMA((3,)),
            pltpu.SemaphoreType.DMA((3,)),
        ],
        compiler_params=pltpu.CompilerParams(collective_id=0),
    )(Q, K, V, bt, lens)


# device time: 16847 ns/iter; 1.0465x vs baseline; 1.0465x over previous
import jax
import jax.numpy as jnp
from jax import lax
from jax.experimental import pallas as pl
from jax.experimental.pallas import tpu as pltpu

B, H, D, BS = 8, 8, 64, 16
SLOTS = 64
PAGES_LOCAL = 64
T = PAGES_LOCAL * BS
NEG = -1e30


def kernel(Q, K, V, bt, lens):
    def body(q_ref, k_ref, v_ref, bt_ref, lens_ref, out_ref,
             o_buf, w_pgT, send_sem, recv_sem):
        my_x = lax.axis_index("x")
        my_y = lax.axis_index("y")

        barrier = pltpu.get_barrier_semaphore()
        pl.semaphore_signal(
            barrier, inc=1,
            device_id=(1 - my_x, my_y), device_id_type=pl.DeviceIdType.MESH,
        )
        pl.semaphore_wait(barrier, 1)

        p_col = my_x * PAGES_LOCAL + lax.broadcasted_iota(
            jnp.int32, (PAGES_LOCAL, 1), 0
        )
        slot_row = lax.broadcasted_iota(jnp.int32, (1, SLOTS), 1)
        for b in range(B):
            bt_row = bt_ref[b : b + 1, :]
            valid_row = slot_row < lens_ref[b]
            cmp = (bt_row == p_col) & valid_row
            w_pgT[:, b : b + 1] = jnp.sum(
                cmp.astype(jnp.float32), axis=1, keepdims=True
            )
        exp_mat = (
            lax.broadcasted_iota(jnp.int32, (PAGES_LOCAL, T), 0)
            == lax.broadcasted_iota(jnp.int32, (PAGES_LOCAL, T), 1) // BS
        ).astype(jnp.float32)
        w_tokT = lax.dot_general(
            exp_mat, w_pgT[:], (((0,), (0,)), ((), ())),
            preferred_element_type=jnp.float32,
        )
        lnwT = jnp.where(w_tokT > 0.0, jnp.log(w_tokT), NEG)

        scale = D ** -0.5
        for h in range(H):
            q_h = q_ref[:, 0, h, :]
            k_h = k_ref[:, :, h, :].reshape(T, D)
            v_h = v_ref[:, :, h, :].reshape(T, D)
            sT = lax.dot_general(
                k_h, q_h, (((1,), (1,)), ((), ())),
                preferred_element_type=jnp.float32,
            )
            pT = jnp.exp(sT * scale + lnwT)
            v_aug = jnp.concatenate(
                [v_h, jnp.ones((T, 1), jnp.float32)], axis=1
            )
            o_aug = lax.dot_general(
                pT, v_aug, (((0,), (0,)), ((), ())),
                preferred_element_type=jnp.float32,
            )
            o_buf[0, h] = o_aug

        rdma = pltpu.make_async_remote_copy(
            src_ref=o_buf.at[0],
            dst_ref=o_buf.at[1],
            send_sem=send_sem,
            recv_sem=recv_sem,
            device_id=(1 - my_x, my_y),
            device_id_type=pl.DeviceIdType.MESH,
        )
        rdma.start()
        rdma.wait()

        for h in range(H):
            s0 = o_buf[0, h]
            s1 = o_buf[1, h]
            num = s0[:, :D] + s1[:, :D]
            den = s0[:, D : D + 1] + s1[:, D : D + 1]
            out_ref[:, 0, h, :] = num / den

    return pl.pallas_call(
        body,
        out_shape=jax.ShapeDtypeStruct((B, 1, H, D), jnp.float32),
        in_specs=[pl.BlockSpec(memory_space=pltpu.VMEM)] * 4
        + [pl.BlockSpec(memory_space=pltpu.SMEM)],
        out_specs=pl.BlockSpec(memory_space=pltpu.VMEM),
        scratch_shapes=[
            pltpu.VMEM((2, H, B, D + 1), jnp.float32),
            pltpu.VMEM((PAGES_LOCAL, B), jnp.float32),
            pltpu.SemaphoreType.DMA,
            pltpu.SemaphoreType.DMA,
        ],
        compiler_params=pltpu.CompilerParams(collective_id=0),
    )(Q, K, V, bt, lens)
